# baseline (speedup 1.0000x reference)
import functools

import jax
import jax.numpy as jnp
from jax import lax
from jax.experimental import pallas as pl
from jax.experimental.pallas import tpu as pltpu
from jax.experimental.pallas import tpu_sc as plsc

N_NODES = 10000
N_EDGES = 320000
D_FEAT = 128
D_EDGE = 16
D_HID = 128

NC = 2
NS = 16
NW = NC * NS
C = 80
EDGES_PER_W = N_EDGES // NW
CHUNKS = EDGES_PER_W // C
N_PAD = 10240
ROWS_PER_TILE = N_PAD // NS
ZROWS = 128


def _proj_body(edges_ref, we_ref, be_ref, nodes_ref, wr_ref, ws_ref,
               eout_ref, pr_ref, ps_ref):
    eout_ref[:] = (
        jnp.dot(edges_ref[:], we_ref[:], preferred_element_type=jnp.float32)
        + be_ref[:]
    )
    n = nodes_ref[:]
    pr_ref[:] = jnp.dot(n, wr_ref[:], preferred_element_type=jnp.float32)
    ps_ref[:] = jnp.dot(n, ws_ref[:], preferred_element_type=jnp.float32)


def _sc_edge_body(eproj_hbm, pr_hbm, ps_hbm, send_hbm, recv_hbm,
                  edges_out_hbm, agg_out_hbm,
                  ridx0, ridx1, sidx0, sidx1, acc0, acc1,
                  zbuf_v, agg_sh,
                  lsem0, lsem1, gsem0, gsem1, osem0, osem1, csem0, csem1):
    c = lax.axis_index("c")
    s = lax.axis_index("s")
    w = s * NC + c
    ridx = (ridx0, ridx1)
    sidx = (sidx0, sidx1)
    acc = (acc0, acc1)
    lsem = (lsem0, lsem1)
    gsem = (gsem0, gsem1)
    osem = (osem0, osem1)
    csem = (csem0, csem1)

    zero16 = jnp.zeros((16,), jnp.float32)

    def _zrow(i, carry):
        for j in range(8):
            zbuf_v[i, pl.ds(j * 16, 16)] = zero16
        return carry

    lax.fori_loop(0, ZROWS, _zrow, 0)
    for k in range(ROWS_PER_TILE // ZROWS):
        pltpu.sync_copy(
            zbuf_v, agg_sh.at[pl.ds(s * ROWS_PER_TILE + k * ZROWS, ZROWS), :]
        )
    plsc.subcore_barrier()

    def _load(i, b):
        base = w * EDGES_PER_W + i * C
        pltpu.async_copy(recv_hbm.at[pl.ds(base, C)], ridx[b], lsem[b])
        pltpu.async_copy(send_hbm.at[pl.ds(base, C)], sidx[b], lsem[b])
        pltpu.async_copy(eproj_hbm.at[pl.ds(base, C), :], acc[b], lsem[b])

    def _gathers(b):
        pltpu.make_async_copy(recv_hbm.at[pl.ds(0, C)], ridx[b], lsem[b]).wait()
        pltpu.make_async_copy(send_hbm.at[pl.ds(0, C)], sidx[b], lsem[b]).wait()
        pltpu.make_async_copy(
            eproj_hbm.at[pl.ds(0, C), :], acc[b], lsem[b]
        ).wait()
        pltpu.async_copy(pr_hbm.at[ridx[b]], acc[b], gsem[b], add=True)
        pltpu.async_copy(ps_hbm.at[sidx[b]], acc[b], gsem[b], add=True)

    def _finish(i, b):
        base = w * EDGES_PER_W + i * C
        pltpu.make_async_copy(pr_hbm.at[ridx[b]], acc[b], gsem[b]).wait()
        pltpu.make_async_copy(ps_hbm.at[sidx[b]], acc[b], gsem[b]).wait()

        def _relu_row(e, cc):
            for j in range(8):
                x = acc[b][e, pl.ds(j * 16, 16)]
                acc[b][e, pl.ds(j * 16, 16)] = jnp.maximum(x, 0.0)
            return cc

        lax.fori_loop(0, C, _relu_row, 0)
        pltpu.async_copy(acc[b], edges_out_hbm.at[pl.ds(base, C), :], osem[b])
        pltpu.async_copy(acc[b], agg_sh.at[ridx[b]], csem[b], add=True)

    def _drain(b):
        pltpu.make_async_copy(
            acc[b], edges_out_hbm.at[pl.ds(0, C), :], osem[b]
        ).wait()
        pltpu.make_async_copy(acc[b], agg_sh.at[ridx[b]], csem[b]).wait()

    _load(0, 0)
    _gathers(0)
    _load(1, 1)
    _finish(0, 0)

    def _body(jj, carry):
        i1 = 1 + 2 * jj
        _gathers(1)
        _drain(0)
        _load(i1 + 1, 0)
        _finish(i1, 1)
        _gathers(0)
        _drain(1)

        @pl.when(i1 + 2 < CHUNKS)
        def _():
            _load(i1 + 2, 1)

        _finish(i1 + 1, 0)
        return carry

    lax.fori_loop(0, (CHUNKS - 1) // 2, _body, 0)
    _drain(0)

    plsc.subcore_barrier()
    pltpu.sync_copy(
        agg_sh.at[pl.ds(s * ROWS_PER_TILE, ROWS_PER_TILE), :],
        agg_out_hbm.at[c, pl.ds(s * ROWS_PER_TILE, ROWS_PER_TILE), :],
    )


def _node_global_body(agg2_ref, nodes_ref, wna_ref, wnn_ref, bn_ref,
                      wga_ref, wgb_ref, bg_ref, nodes_out_ref, glob_out_ref):
    agg = agg2_ref[0] + agg2_ref[1]
    h = jnp.maximum(
        jnp.dot(agg, wna_ref[:], preferred_element_type=jnp.float32)
        + jnp.dot(nodes_ref[:], wnn_ref[:], preferred_element_type=jnp.float32)
        + bn_ref[:],
        0.0,
    )
    nodes_out_ref[:] = h
    se = jnp.sum(agg, axis=0, keepdims=True)
    sn = jnp.sum(h, axis=0, keepdims=True)
    g = (
        jnp.dot(se, wga_ref[:], preferred_element_type=jnp.float32)
        + jnp.dot(sn, wgb_ref[:], preferred_element_type=jnp.float32)
        + bg_ref[:]
    )
    glob_out_ref[:] = jnp.maximum(g, 0.0)


def kernel(nodes, edges, senders, receivers, We, be, Wn, bn, Wg, bg):
    we_e = We[:D_EDGE]
    we_r = We[D_EDGE:D_EDGE + D_FEAT]
    we_s = We[D_EDGE + D_FEAT:]

    eblk = 2560
    nblk = 80
    e_proj, pr, ps = pl.pallas_call(
        _proj_body,
        grid=(N_EDGES // eblk,),
        in_specs=[
            pl.BlockSpec((eblk, D_EDGE), lambda i: (i, 0)),
            pl.BlockSpec((D_EDGE, D_HID), lambda i: (0, 0)),
            pl.BlockSpec((1, D_HID), lambda i: (0, 0)),
            pl.BlockSpec((nblk, D_FEAT), lambda i: (i, 0)),
            pl.BlockSpec((D_FEAT, D_HID), lambda i: (0, 0)),
            pl.BlockSpec((D_FEAT, D_HID), lambda i: (0, 0)),
        ],
        out_specs=[
            pl.BlockSpec((eblk, D_HID), lambda i: (i, 0)),
            pl.BlockSpec((nblk, D_HID), lambda i: (i, 0)),
            pl.BlockSpec((nblk, D_HID), lambda i: (i, 0)),
        ],
        out_shape=[
            jax.ShapeDtypeStruct((N_EDGES, D_HID), jnp.float32),
            jax.ShapeDtypeStruct((N_NODES, D_HID), jnp.float32),
            jax.ShapeDtypeStruct((N_NODES, D_HID), jnp.float32),
        ],
    )(edges, we_e, be.reshape(1, D_HID), nodes, we_r, we_s)

    sc_edge = pl.kernel(
        _sc_edge_body,
        out_type=(
            jax.ShapeDtypeStruct((N_EDGES, D_HID), jnp.float32),
            jax.ShapeDtypeStruct((NC, N_PAD, D_HID), jnp.float32),
        ),
        mesh=plsc.VectorSubcoreMesh(core_axis_name="c", subcore_axis_name="s"),
        scratch_types=[
            pltpu.VMEM((C,), jnp.int32),
            pltpu.VMEM((C,), jnp.int32),
            pltpu.VMEM((C,), jnp.int32),
            pltpu.VMEM((C,), jnp.int32),
            pltpu.VMEM((C, D_HID), jnp.float32),
            pltpu.VMEM((C, D_HID), jnp.float32),
            pltpu.VMEM((ZROWS, D_HID), jnp.float32),
            pltpu.VMEM_SHARED((N_PAD, D_HID), jnp.float32),
        ] + [pltpu.SemaphoreType.DMA] * 8,
    )
    new_edges, agg2 = sc_edge(e_proj, pr, ps, senders, receivers)
    agg2 = agg2[:, :N_NODES, :]

    new_nodes, new_globals = pl.pallas_call(
        _node_global_body,
        grid=(1,),
        in_specs=[
            pl.BlockSpec((NC, N_NODES, D_HID), lambda i: (0, 0, 0)),
            pl.BlockSpec((N_NODES, D_FEAT), lambda i: (0, 0)),
            pl.BlockSpec((D_HID, D_HID), lambda i: (0, 0)),
            pl.BlockSpec((D_FEAT, D_HID), lambda i: (0, 0)),
            pl.BlockSpec((1, D_HID), lambda i: (0, 0)),
            pl.BlockSpec((D_HID, D_HID), lambda i: (0, 0)),
            pl.BlockSpec((D_HID, D_HID), lambda i: (0, 0)),
            pl.BlockSpec((1, D_HID), lambda i: (0, 0)),
        ],
        out_specs=[
            pl.BlockSpec((N_NODES, D_HID), lambda i: (0, 0)),
            pl.BlockSpec((1, D_HID), lambda i: (0, 0)),
        ],
        out_shape=[
            jax.ShapeDtypeStruct((N_NODES, D_HID), jnp.float32),
            jax.ShapeDtypeStruct((1, D_HID), jnp.float32),
        ],
    )(
        agg2, nodes, Wn[:D_HID], Wn[D_HID:], bn.reshape(1, D_HID),
        Wg[:D_HID], Wg[D_HID:], bg.reshape(1, D_HID),
    )

    return (new_edges, new_nodes, new_globals)


# speedup vs baseline: 1.0866x; 1.0866x over previous
import functools

import jax
import jax.numpy as jnp
from jax import lax
from jax.experimental import pallas as pl
from jax.experimental.pallas import tpu as pltpu
from jax.experimental.pallas import tpu_sc as plsc

N_NODES = 10000
N_EDGES = 320000
D_FEAT = 128
D_EDGE = 16
D_HID = 128

NC = 2
NS = 16
NW = NC * NS
C = 80
EDGES_PER_W = N_EDGES // NW
CHUNKS = EDGES_PER_W // C
N_PAD = 10240
ROWS_PER_TILE = N_PAD // NS
ZROWS = 128


def _proj_body(edges_ref, we_ref, be_ref, nodes_ref, wr_ref, ws_ref,
               eout_ref, pr_ref, ps_ref):
    eout_ref[:] = (
        jnp.dot(edges_ref[:], we_ref[:], preferred_element_type=jnp.float32)
        + be_ref[:]
    )
    n = nodes_ref[:]
    pr_ref[:] = jnp.dot(n, wr_ref[:], preferred_element_type=jnp.float32)
    ps_ref[:] = jnp.dot(n, ws_ref[:], preferred_element_type=jnp.float32)


def _sc_edge_body(eproj_hbm, pr_hbm, ps_hbm, send_hbm, recv_hbm,
                  edges_out_hbm, agg_out_hbm,
                  ridx0, ridx1, sidx0, sidx1, acc0, acc1,
                  zbuf_v, agg_sh,
                  lsem0, lsem1, gsem0, gsem1, osem0, osem1, csem0, csem1):
    c = lax.axis_index("c")
    s = lax.axis_index("s")
    w = s * NC + c
    ridx = (ridx0, ridx1)
    sidx = (sidx0, sidx1)
    acc = (acc0, acc1)
    lsem = (lsem0, lsem1)
    gsem = (gsem0, gsem1)
    osem = (osem0, osem1)
    csem = (csem0, csem1)

    zero16 = jnp.zeros((16,), jnp.float32)

    def _zrow(i, carry):
        for j in range(8):
            zbuf_v[i, pl.ds(j * 16, 16)] = zero16
        return carry

    lax.fori_loop(0, ZROWS, _zrow, 0)
    for k in range(ROWS_PER_TILE // ZROWS):
        pltpu.sync_copy(
            zbuf_v, agg_sh.at[pl.ds(s * ROWS_PER_TILE + k * ZROWS, ZROWS), :]
        )
    plsc.subcore_barrier()

    def _load(i, b):
        base = w * EDGES_PER_W + i * C
        pltpu.async_copy(recv_hbm.at[pl.ds(base, C)], ridx[b], lsem[b])
        pltpu.async_copy(send_hbm.at[pl.ds(base, C)], sidx[b], lsem[b])
        pltpu.async_copy(eproj_hbm.at[pl.ds(base, C), :], acc[b], lsem[b])

    def _gathers(b):
        pltpu.make_async_copy(recv_hbm.at[pl.ds(0, C)], ridx[b], lsem[b]).wait()
        pltpu.make_async_copy(send_hbm.at[pl.ds(0, C)], sidx[b], lsem[b]).wait()
        pltpu.make_async_copy(
            eproj_hbm.at[pl.ds(0, C), :], acc[b], lsem[b]
        ).wait()
        pltpu.async_copy(pr_hbm.at[ridx[b]], acc[b], gsem[b], add=True)
        pltpu.async_copy(ps_hbm.at[sidx[b]], acc[b], gsem[b], add=True)

    def _finish(i, b):
        base = w * EDGES_PER_W + i * C
        pltpu.make_async_copy(pr_hbm.at[ridx[b]], acc[b], gsem[b]).wait()
        pltpu.make_async_copy(ps_hbm.at[sidx[b]], acc[b], gsem[b]).wait()

        def _relu_row(e, cc):
            for j in range(8):
                x = acc[b][e, pl.ds(j * 16, 16)]
                acc[b][e, pl.ds(j * 16, 16)] = jnp.maximum(x, 0.0)
            return cc

        lax.fori_loop(0, C, _relu_row, 0)
        pltpu.async_copy(acc[b], edges_out_hbm.at[pl.ds(base, C), :], osem[b])
        pltpu.async_copy(acc[b], agg_sh.at[ridx[b]], csem[b], add=True)

    def _drain(b):
        pltpu.make_async_copy(
            acc[b], edges_out_hbm.at[pl.ds(0, C), :], osem[b]
        ).wait()
        pltpu.make_async_copy(acc[b], agg_sh.at[ridx[b]], csem[b]).wait()

    _load(0, 0)
    _gathers(0)
    _load(1, 1)
    _finish(0, 0)

    def _body(jj, carry):
        i1 = 1 + 2 * jj
        _gathers(1)
        _drain(0)
        _load(i1 + 1, 0)
        _finish(i1, 1)
        _gathers(0)
        _drain(1)

        @pl.when(i1 + 2 < CHUNKS)
        def _():
            _load(i1 + 2, 1)

        _finish(i1 + 1, 0)
        return carry

    lax.fori_loop(0, (CHUNKS - 1) // 2, _body, 0)
    _drain(0)

    plsc.subcore_barrier()
    pltpu.sync_copy(
        agg_sh.at[pl.ds(s * ROWS_PER_TILE, ROWS_PER_TILE), :],
        agg_out_hbm.at[c, pl.ds(s * ROWS_PER_TILE, ROWS_PER_TILE), :],
    )


def _node_global_body(agg2_ref, nodes_ref, wna_ref, wnn_ref, bn_ref,
                      wga_ref, wgb_ref, bg_ref, nodes_out_ref, glob_out_ref):
    agg = agg2_ref[0] + agg2_ref[1]
    h = jnp.maximum(
        jnp.dot(agg, wna_ref[:], preferred_element_type=jnp.float32)
        + jnp.dot(nodes_ref[:], wnn_ref[:], preferred_element_type=jnp.float32)
        + bn_ref[:],
        0.0,
    )
    nodes_out_ref[:] = h
    se = jnp.sum(agg, axis=0, keepdims=True)
    sn = jnp.sum(h, axis=0, keepdims=True)
    g = (
        jnp.dot(se, wga_ref[:], preferred_element_type=jnp.float32)
        + jnp.dot(sn, wgb_ref[:], preferred_element_type=jnp.float32)
        + bg_ref[:]
    )
    glob_out_ref[:] = jnp.maximum(g, 0.0)


def kernel(nodes, edges, senders, receivers, We, be, Wn, bn, Wg, bg):
    we_e = We[:D_EDGE]
    we_r = We[D_EDGE:D_EDGE + D_FEAT]
    we_s = We[D_EDGE + D_FEAT:]

    eblk = 12800
    nblk = 400
    e_proj, pr, ps = pl.pallas_call(
        _proj_body,
        grid=(N_EDGES // eblk,),
        in_specs=[
            pl.BlockSpec((eblk, D_EDGE), lambda i: (i, 0)),
            pl.BlockSpec((D_EDGE, D_HID), lambda i: (0, 0)),
            pl.BlockSpec((1, D_HID), lambda i: (0, 0)),
            pl.BlockSpec((nblk, D_FEAT), lambda i: (i, 0)),
            pl.BlockSpec((D_FEAT, D_HID), lambda i: (0, 0)),
            pl.BlockSpec((D_FEAT, D_HID), lambda i: (0, 0)),
        ],
        out_specs=[
            pl.BlockSpec((eblk, D_HID), lambda i: (i, 0)),
            pl.BlockSpec((nblk, D_HID), lambda i: (i, 0)),
            pl.BlockSpec((nblk, D_HID), lambda i: (i, 0)),
        ],
        out_shape=[
            jax.ShapeDtypeStruct((N_EDGES, D_HID), jnp.float32),
            jax.ShapeDtypeStruct((N_NODES, D_HID), jnp.float32),
            jax.ShapeDtypeStruct((N_NODES, D_HID), jnp.float32),
        ],
    )(edges, we_e, be.reshape(1, D_HID), nodes, we_r, we_s)

    sc_edge = pl.kernel(
        _sc_edge_body,
        out_type=(
            jax.ShapeDtypeStruct((N_EDGES, D_HID), jnp.float32),
            jax.ShapeDtypeStruct((NC, N_PAD, D_HID), jnp.float32),
        ),
        mesh=plsc.VectorSubcoreMesh(core_axis_name="c", subcore_axis_name="s"),
        scratch_types=[
            pltpu.VMEM((C,), jnp.int32),
            pltpu.VMEM((C,), jnp.int32),
            pltpu.VMEM((C,), jnp.int32),
            pltpu.VMEM((C,), jnp.int32),
            pltpu.VMEM((C, D_HID), jnp.float32),
            pltpu.VMEM((C, D_HID), jnp.float32),
            pltpu.VMEM((ZROWS, D_HID), jnp.float32),
            pltpu.VMEM_SHARED((N_PAD, D_HID), jnp.float32),
        ] + [pltpu.SemaphoreType.DMA] * 8,
    )
    new_edges, agg2 = sc_edge(e_proj, pr, ps, senders, receivers)
    agg2 = agg2[:, :N_NODES, :]

    new_nodes, new_globals = pl.pallas_call(
        _node_global_body,
        grid=(1,),
        in_specs=[
            pl.BlockSpec((NC, N_NODES, D_HID), lambda i: (0, 0, 0)),
            pl.BlockSpec((N_NODES, D_FEAT), lambda i: (0, 0)),
            pl.BlockSpec((D_HID, D_HID), lambda i: (0, 0)),
            pl.BlockSpec((D_FEAT, D_HID), lambda i: (0, 0)),
            pl.BlockSpec((1, D_HID), lambda i: (0, 0)),
            pl.BlockSpec((D_HID, D_HID), lambda i: (0, 0)),
            pl.BlockSpec((D_HID, D_HID), lambda i: (0, 0)),
            pl.BlockSpec((1, D_HID), lambda i: (0, 0)),
        ],
        out_specs=[
            pl.BlockSpec((N_NODES, D_HID), lambda i: (0, 0)),
            pl.BlockSpec((1, D_HID), lambda i: (0, 0)),
        ],
        out_shape=[
            jax.ShapeDtypeStruct((N_NODES, D_HID), jnp.float32),
            jax.ShapeDtypeStruct((1, D_HID), jnp.float32),
        ],
    )(
        agg2, nodes, Wn[:D_HID], Wn[D_HID:], bn.reshape(1, D_HID),
        Wg[:D_HID], Wg[D_HID:], bg.reshape(1, D_HID),
    )

    return (new_edges, new_nodes, new_globals)


# speedup vs baseline: 1.1800x; 1.0860x over previous
import functools

import jax
import jax.numpy as jnp
from jax import lax
from jax.experimental import pallas as pl
from jax.experimental.pallas import tpu as pltpu
from jax.experimental.pallas import tpu_sc as plsc

N_NODES = 10000
N_EDGES = 320000
D_FEAT = 128
D_EDGE = 16
D_HID = 128

NC = 2
NS = 16
NW = NC * NS
C = 80
NBUF = 4
EDGES_PER_W = N_EDGES // NW
CHUNKS = EDGES_PER_W // C
N_PAD = 10240
ROWS_PER_TILE = N_PAD // NS
ZROWS = 32


def _proj_body(edges_ref, we_ref, be_ref, nodes_ref, wr_ref, ws_ref,
               eout_ref, pr_ref, ps_ref):
    eout_ref[:] = (
        jnp.dot(edges_ref[:], we_ref[:], preferred_element_type=jnp.float32)
        + be_ref[:]
    )
    n = nodes_ref[:]
    pr_ref[:] = jnp.dot(n, wr_ref[:], preferred_element_type=jnp.float32)
    ps_ref[:] = jnp.dot(n, ws_ref[:], preferred_element_type=jnp.float32)


def _sc_edge_body(eproj_hbm, pr_hbm, ps_hbm, send_hbm, recv_hbm,
                  edges_out_hbm, agg_out_hbm,
                  ridx0, ridx1, ridx2, ridx3, sidx0, sidx1, sidx2, sidx3,
                  acc0, acc1, acc2, acc3, zbuf_v, agg_sh,
                  lsem0, lsem1, lsem2, lsem3, gsem0, gsem1, gsem2, gsem3,
                  osem0, osem1, osem2, osem3, csem0, csem1, csem2, csem3):
    c = lax.axis_index("c")
    s = lax.axis_index("s")
    w = s * NC + c
    ridx = (ridx0, ridx1, ridx2, ridx3)
    sidx = (sidx0, sidx1, sidx2, sidx3)
    acc = (acc0, acc1, acc2, acc3)
    lsem = (lsem0, lsem1, lsem2, lsem3)
    gsem = (gsem0, gsem1, gsem2, gsem3)
    osem = (osem0, osem1, osem2, osem3)
    csem = (csem0, csem1, csem2, csem3)

    zero16 = jnp.zeros((16,), jnp.float32)

    def _zrow(i, carry):
        for j in range(8):
            zbuf_v[i, pl.ds(j * 16, 16)] = zero16
        return carry

    lax.fori_loop(0, ZROWS, _zrow, 0)
    for k in range(ROWS_PER_TILE // ZROWS):
        pltpu.sync_copy(
            zbuf_v, agg_sh.at[pl.ds(s * ROWS_PER_TILE + k * ZROWS, ZROWS), :]
        )
    plsc.subcore_barrier()

    def _load(i, b):
        base = w * EDGES_PER_W + i * C
        pltpu.async_copy(recv_hbm.at[pl.ds(base, C)], ridx[b], lsem[b])
        pltpu.async_copy(send_hbm.at[pl.ds(base, C)], sidx[b], lsem[b])
        pltpu.async_copy(eproj_hbm.at[pl.ds(base, C), :], acc[b], lsem[b])

    def _gathers(b):
        pltpu.make_async_copy(recv_hbm.at[pl.ds(0, C)], ridx[b], lsem[b]).wait()
        pltpu.make_async_copy(send_hbm.at[pl.ds(0, C)], sidx[b], lsem[b]).wait()
        pltpu.make_async_copy(
            eproj_hbm.at[pl.ds(0, C), :], acc[b], lsem[b]
        ).wait()
        pltpu.async_copy(pr_hbm.at[ridx[b]], acc[b], gsem[b], add=True)
        pltpu.async_copy(ps_hbm.at[sidx[b]], acc[b], gsem[b], add=True)

    def _finish(i, b):
        base = w * EDGES_PER_W + i * C
        pltpu.make_async_copy(pr_hbm.at[ridx[b]], acc[b], gsem[b]).wait()
        pltpu.make_async_copy(ps_hbm.at[sidx[b]], acc[b], gsem[b]).wait()

        def _relu_row(e, cc):
            for j in range(8):
                x = acc[b][e, pl.ds(j * 16, 16)]
                acc[b][e, pl.ds(j * 16, 16)] = jnp.maximum(x, 0.0)
            return cc

        lax.fori_loop(0, C, _relu_row, 0)
        pltpu.async_copy(acc[b], edges_out_hbm.at[pl.ds(base, C), :], osem[b])
        pltpu.async_copy(acc[b], agg_sh.at[ridx[b]], csem[b], add=True)

    def _drain(b):
        pltpu.make_async_copy(
            acc[b], edges_out_hbm.at[pl.ds(0, C), :], osem[b]
        ).wait()
        pltpu.make_async_copy(acc[b], agg_sh.at[ridx[b]], csem[b]).wait()

    _load(0, 0)
    _load(1, 1)
    _gathers(0)

    for r in range(4):
        i = r
        _gathers((r + 1) % NBUF)
        if i >= 2:
            _drain((r + 2) % NBUF)
        _load(i + 2, (r + 2) % NBUF)
        _finish(i, r)

    def _body(jj, carry):
        i0 = 4 * jj
        for r in range(4):
            i = i0 + r
            _gathers((r + 1) % NBUF)
            _drain((r + 2) % NBUF)

            @pl.when(i + 2 < CHUNKS)
            def _():
                _load(i + 2, (r + 2) % NBUF)

            _finish(i, r)
        return carry

    lax.fori_loop(1, 31, _body, 0)

    _drain(2)
    _finish(124, 0)
    _drain(3)
    _drain(0)

    plsc.subcore_barrier()
    pltpu.sync_copy(
        agg_sh.at[pl.ds(s * ROWS_PER_TILE, ROWS_PER_TILE), :],
        agg_out_hbm.at[c, pl.ds(s * ROWS_PER_TILE, ROWS_PER_TILE), :],
    )


def _node_global_body(agg2_ref, nodes_ref, wna_ref, wnn_ref, bn_ref,
                      wga_ref, wgb_ref, bg_ref, nodes_out_ref, glob_out_ref):
    agg = agg2_ref[0] + agg2_ref[1]
    h = jnp.maximum(
        jnp.dot(agg, wna_ref[:], preferred_element_type=jnp.float32)
        + jnp.dot(nodes_ref[:], wnn_ref[:], preferred_element_type=jnp.float32)
        + bn_ref[:],
        0.0,
    )
    nodes_out_ref[:] = h
    se = jnp.sum(agg, axis=0, keepdims=True)
    sn = jnp.sum(h, axis=0, keepdims=True)
    g = (
        jnp.dot(se, wga_ref[:], preferred_element_type=jnp.float32)
        + jnp.dot(sn, wgb_ref[:], preferred_element_type=jnp.float32)
        + bg_ref[:]
    )
    glob_out_ref[:] = jnp.maximum(g, 0.0)


def kernel(nodes, edges, senders, receivers, We, be, Wn, bn, Wg, bg):
    we_e = We[:D_EDGE]
    we_r = We[D_EDGE:D_EDGE + D_FEAT]
    we_s = We[D_EDGE + D_FEAT:]

    eblk = 12800
    nblk = 400
    e_proj, pr, ps = pl.pallas_call(
        _proj_body,
        grid=(N_EDGES // eblk,),
        in_specs=[
            pl.BlockSpec((eblk, D_EDGE), lambda i: (i, 0)),
            pl.BlockSpec((D_EDGE, D_HID), lambda i: (0, 0)),
            pl.BlockSpec((1, D_HID), lambda i: (0, 0)),
            pl.BlockSpec((nblk, D_FEAT), lambda i: (i, 0)),
            pl.BlockSpec((D_FEAT, D_HID), lambda i: (0, 0)),
            pl.BlockSpec((D_FEAT, D_HID), lambda i: (0, 0)),
        ],
        out_specs=[
            pl.BlockSpec((eblk, D_HID), lambda i: (i, 0)),
            pl.BlockSpec((nblk, D_HID), lambda i: (i, 0)),
            pl.BlockSpec((nblk, D_HID), lambda i: (i, 0)),
        ],
        out_shape=[
            jax.ShapeDtypeStruct((N_EDGES, D_HID), jnp.float32),
            jax.ShapeDtypeStruct((N_NODES, D_HID), jnp.float32),
            jax.ShapeDtypeStruct((N_NODES, D_HID), jnp.float32),
        ],
    )(edges, we_e, be.reshape(1, D_HID), nodes, we_r, we_s)

    sc_edge = pl.kernel(
        _sc_edge_body,
        out_type=(
            jax.ShapeDtypeStruct((N_EDGES, D_HID), jnp.float32),
            jax.ShapeDtypeStruct((NC, N_PAD, D_HID), jnp.float32),
        ),
        mesh=plsc.VectorSubcoreMesh(core_axis_name="c", subcore_axis_name="s"),
        scratch_types=(
            [pltpu.VMEM((C,), jnp.int32)] * 8
            + [pltpu.VMEM((C, D_HID), jnp.float32)] * 4
            + [
                pltpu.VMEM((ZROWS, D_HID), jnp.float32),
                pltpu.VMEM_SHARED((N_PAD, D_HID), jnp.float32),
            ]
            + [pltpu.SemaphoreType.DMA] * 16
        ),
    )
    new_edges, agg2 = sc_edge(e_proj, pr, ps, senders, receivers)
    agg2 = agg2[:, :N_NODES, :]

    new_nodes, new_globals = pl.pallas_call(
        _node_global_body,
        grid=(1,),
        in_specs=[
            pl.BlockSpec((NC, N_NODES, D_HID), lambda i: (0, 0, 0)),
            pl.BlockSpec((N_NODES, D_FEAT), lambda i: (0, 0)),
            pl.BlockSpec((D_HID, D_HID), lambda i: (0, 0)),
            pl.BlockSpec((D_FEAT, D_HID), lambda i: (0, 0)),
            pl.BlockSpec((1, D_HID), lambda i: (0, 0)),
            pl.BlockSpec((D_HID, D_HID), lambda i: (0, 0)),
            pl.BlockSpec((D_HID, D_HID), lambda i: (0, 0)),
            pl.BlockSpec((1, D_HID), lambda i: (0, 0)),
        ],
        out_specs=[
            pl.BlockSpec((N_NODES, D_HID), lambda i: (0, 0)),
            pl.BlockSpec((1, D_HID), lambda i: (0, 0)),
        ],
        out_shape=[
            jax.ShapeDtypeStruct((N_NODES, D_HID), jnp.float32),
            jax.ShapeDtypeStruct((1, D_HID), jnp.float32),
        ],
    )(
        agg2, nodes, Wn[:D_HID], Wn[D_HID:], bn.reshape(1, D_HID),
        Wg[:D_HID], Wg[D_HID:], bg.reshape(1, D_HID),
    )

    return (new_edges, new_nodes, new_globals)
